# baseline (speedup 1.0000x reference)
import functools

import jax
import jax.numpy as jnp
from jax import lax
from jax.experimental import pallas as pl
from jax.experimental.pallas import tpu as pltpu
from jax.experimental.pallas import tpu_sc as plsc

BATCH = 16384
SEQ = 200
DIM = 64
NCLS = 4
PDIM = 16

NC = 2
NS = 16
NW = NC * NS
BPW = BATCH // NW
CL = 8
CI = CL * SEQ
NCLUST = BPW // CL
NBUF = 2
CHUNKS = [(o, min(128, CI - o)) for o in range(0, CI, 128)]

VOCAB = 1000000
TC_BM = 4000

_mesh = plsc.VectorSubcoreMesh(core_axis_name="c", subcore_axis_name="s")


def _proj_body(t_ref, w_ref, o_ref):
    o_ref[...] = jnp.dot(t_ref[...], w_ref[...],
                         preferred_element_type=jnp.float32)


_project = pl.pallas_call(
    _proj_body,
    grid=(VOCAB // TC_BM,),
    in_specs=[
        pl.BlockSpec((TC_BM, DIM), lambda i: (i, 0)),
        pl.BlockSpec((DIM, PDIM), lambda i: (0, 0)),
    ],
    out_specs=pl.BlockSpec((TC_BM, PDIM), lambda i: (i, 0)),
    out_shape=jax.ShapeDtypeStruct((VOCAB, PDIM), jnp.float32),
)


@functools.partial(
    pl.kernel,
    mesh=_mesh,
    compiler_params=pltpu.CompilerParams(
        use_tc_tiling_on_sc=False, needs_layout_passes=False),
    out_type=jax.ShapeDtypeStruct((BATCH * NCLS,), jnp.float32),
    scratch_types=[
        pltpu.VMEM((NBUF, CI), jnp.int32),
        pltpu.VMEM((NBUF, CI, PDIM), jnp.float32),
        pltpu.VMEM((BPW * NCLS,), jnp.float32),
        pltpu.VMEM((16,), jnp.float32),
        pltpu.VMEM((4 * PDIM,), jnp.float32),
        pltpu.SemaphoreType.DMA,
    ],
)
def _bag_kernel(text_hbm, ptab_hbm, fcb_hbm, out_hbm,
                idx_v, rows_v, out_v, fcb_v, stage_v, sem):
    wid = lax.axis_index("s") * NC + lax.axis_index("c")
    base = wid * BPW

    pltpu.sync_copy(fcb_hbm, fcb_v)
    inv = jnp.float32(1.0 / SEQ)
    bias = fcb_v[...]
    lane = lax.iota(jnp.int32, 16)
    gidx = (lane // 4) * PDIM + (lane % 4)

    def fetch(g, k):
        row0 = base + g * CL
        pltpu.sync_copy(text_hbm.at[pl.ds(row0 * SEQ, CI)], idx_v.at[k])
        for off, ln in CHUNKS:
            pltpu.async_copy(
                ptab_hbm.at[idx_v.at[k, pl.ds(off, ln)]],
                rows_v.at[k, pl.ds(off, ln)],
                sem,
            )

    def drain(k):
        for off, ln in CHUNKS:
            pltpu.make_async_copy(
                ptab_hbm.at[idx_v.at[k, pl.ds(off, ln)]],
                rows_v.at[k, pl.ds(off, ln)],
                sem,
            ).wait()

    for g in range(NBUF):
        fetch(g, g)

    def cluster_body(g, carry):
        k = lax.rem(g, NBUF)
        drain(k)

        for q in range(CL // 4):
            for b in range(4):
                r0 = (q * 4 + b) * SEQ

                def acc_body(i, acc):
                    return acc + rows_v[k, r0 + i, pl.ds(0, 16)]

                acc = lax.fori_loop(0, SEQ, acc_body,
                                    jnp.zeros((16,), jnp.float32))
                stage_v[pl.ds(b * PDIM, 16)] = acc

            res = plsc.load_gather(stage_v, [gidx])
            out_v[pl.ds((g * CL + q * 4) * NCLS, 16)] = res * inv + bias

        gn = lax.min(g + NBUF, NCLUST - 1)
        fetch(gn, k)
        return carry

    lax.fori_loop(0, NCLUST, cluster_body, 0)
    drain(lax.rem(NCLUST, NBUF))
    drain(lax.rem(NCLUST + 1, NBUF))
    pltpu.sync_copy(out_v, out_hbm.at[pl.ds(base * NCLS, BPW * NCLS)])


def kernel(text, emb_table, fc_w, fc_b):
    text_flat = text.reshape(-1).astype(jnp.int32)
    wpad = jnp.zeros((DIM, PDIM), jnp.float32).at[:, :NCLS].set(
        fc_w.astype(jnp.float32).T)
    ptab = _project(emb_table, wpad)
    fcb_tiled = jnp.tile(fc_b.astype(jnp.float32), 4)
    out = _bag_kernel(text_flat, ptab, fcb_tiled)
    return out.reshape(BATCH, NCLS)


# speedup vs baseline: 1.2376x; 1.2376x over previous
import functools

import jax
import jax.numpy as jnp
from jax import lax
from jax.experimental import pallas as pl
from jax.experimental.pallas import tpu as pltpu
from jax.experimental.pallas import tpu_sc as plsc

BATCH = 16384
SEQ = 200
DIM = 64
NCLS = 4
PDIM = 16

NC = 2
NS = 16
NW = NC * NS
BPW = BATCH // NW
CL = 8
CI = CL * SEQ
NCLUST = BPW // CL
NBUF = 2
CHUNKS = [(o, min(128, CI - o)) for o in range(0, CI, 128)]

VOCAB = 1000000
TC_BM = 4000

_mesh = plsc.VectorSubcoreMesh(core_axis_name="c", subcore_axis_name="s")


def _proj_body(t_ref, w_ref, o_ref):
    o_ref[...] = jnp.dot(t_ref[...], w_ref[...],
                         preferred_element_type=jnp.float32)


_project = pl.pallas_call(
    _proj_body,
    grid=(VOCAB // TC_BM,),
    in_specs=[
        pl.BlockSpec((TC_BM, DIM), lambda i: (i, 0)),
        pl.BlockSpec((DIM, PDIM), lambda i: (0, 0)),
    ],
    out_specs=pl.BlockSpec((TC_BM, PDIM), lambda i: (i, 0)),
    out_shape=jax.ShapeDtypeStruct((VOCAB, PDIM), jnp.float32),
)


@functools.partial(
    pl.kernel,
    mesh=_mesh,
    compiler_params=pltpu.CompilerParams(
        use_tc_tiling_on_sc=False, needs_layout_passes=False),
    out_type=jax.ShapeDtypeStruct((BATCH * NCLS,), jnp.float32),
    scratch_types=[
        pltpu.VMEM((NBUF, CI), jnp.int32),
        pltpu.VMEM((NBUF, CI, PDIM), jnp.float32),
        pltpu.VMEM((BPW * NCLS,), jnp.float32),
        pltpu.VMEM((16,), jnp.float32),
        pltpu.VMEM((4 * PDIM,), jnp.float32),
        pltpu.SemaphoreType.DMA,
    ],
)
def _bag_kernel(text_hbm, ptab_hbm, fcb_hbm, out_hbm,
                idx_v, rows_v, out_v, fcb_v, stage_v, sem):
    wid = lax.axis_index("s") * NC + lax.axis_index("c")
    base = wid * BPW

    pltpu.sync_copy(fcb_hbm, fcb_v)
    inv = jnp.float32(1.0 / SEQ)
    bias = fcb_v[...]
    lane = lax.iota(jnp.int32, 16)
    gidx = (lane // 4) * PDIM + (lane % 4)

    def fetch(g, k):
        row0 = base + g * CL
        pltpu.sync_copy(text_hbm.at[pl.ds(row0 * SEQ, CI)], idx_v.at[k])
        for off, ln in CHUNKS:
            pltpu.async_copy(
                ptab_hbm.at[idx_v.at[k, pl.ds(off, ln)]],
                rows_v.at[k, pl.ds(off, ln)],
                sem,
            )

    def drain(k):
        for off, ln in CHUNKS:
            pltpu.make_async_copy(
                ptab_hbm.at[idx_v.at[k, pl.ds(off, ln)]],
                rows_v.at[k, pl.ds(off, ln)],
                sem,
            ).wait()

    for g in range(NBUF):
        fetch(g, g)

    def cluster_body(g, carry):
        k = lax.rem(g, NBUF)
        drain(k)

        for q in range(CL // 4):
            for b in range(4):
                r0 = (q * 4 + b) * SEQ

                def acc_body(i, accs):
                    a0, a1, a2, a3 = accs
                    r = r0 + i * 8
                    a0 = a0 + rows_v[k, r, pl.ds(0, 16)]
                    a1 = a1 + rows_v[k, r + 1, pl.ds(0, 16)]
                    a2 = a2 + rows_v[k, r + 2, pl.ds(0, 16)]
                    a3 = a3 + rows_v[k, r + 3, pl.ds(0, 16)]
                    a0 = a0 + rows_v[k, r + 4, pl.ds(0, 16)]
                    a1 = a1 + rows_v[k, r + 5, pl.ds(0, 16)]
                    a2 = a2 + rows_v[k, r + 6, pl.ds(0, 16)]
                    a3 = a3 + rows_v[k, r + 7, pl.ds(0, 16)]
                    return (a0, a1, a2, a3)

                z = jnp.zeros((16,), jnp.float32)
                a0, a1, a2, a3 = lax.fori_loop(0, SEQ // 8, acc_body,
                                               (z, z, z, z))
                stage_v[pl.ds(b * PDIM, 16)] = (a0 + a1) + (a2 + a3)

            res = plsc.load_gather(stage_v, [gidx])
            out_v[pl.ds((g * CL + q * 4) * NCLS, 16)] = res * inv + bias

        gn = lax.min(g + NBUF, NCLUST - 1)
        fetch(gn, k)
        return carry

    lax.fori_loop(0, NCLUST, cluster_body, 0)
    drain(lax.rem(NCLUST, NBUF))
    drain(lax.rem(NCLUST + 1, NBUF))
    pltpu.sync_copy(out_v, out_hbm.at[pl.ds(base * NCLS, BPW * NCLS)])


def kernel(text, emb_table, fc_w, fc_b):
    text_flat = text.reshape(-1).astype(jnp.int32)
    wpad = jnp.zeros((DIM, PDIM), jnp.float32).at[:, :NCLS].set(
        fc_w.astype(jnp.float32).T)
    ptab = _project(emb_table, wpad)
    fcb_tiled = jnp.tile(fc_b.astype(jnp.float32), 4)
    out = _bag_kernel(text_flat, ptab, fcb_tiled)
    return out.reshape(BATCH, NCLS)


# speedup vs baseline: 1.4895x; 1.2035x over previous
import functools

import jax
import jax.numpy as jnp
from jax import lax
from jax.experimental import pallas as pl
from jax.experimental.pallas import tpu as pltpu
from jax.experimental.pallas import tpu_sc as plsc

BATCH = 16384
SEQ = 200
DIM = 64
NCLS = 4
PDIM = 8
PT = 16

NC = 2
NS = 16
NW = NC * NS
BPW = BATCH // NW
CL = 8
CI = CL * SEQ
NCLUST = BPW // CL
NBUF = 2
CHUNKS = [(o, min(128, CI - o)) for o in range(0, CI, 128)]

VOCAB = 1000000
TC_BN = 16384
TC_GRID = -(-VOCAB // TC_BN)

TSEG = 31256
TCH = 3200
TNCH = -(-TSEG // TCH)
TSTEP = TCH // 16

_mesh = plsc.VectorSubcoreMesh(core_axis_name="c", subcore_axis_name="s")


def _proj_body(t_ref, w_ref, o_ref):
    o_ref[...] = jax.lax.dot_general(w_ref[...], t_ref[...],
                                     (((0,), (0,)), ((), ())),
                                     preferred_element_type=jnp.float32)


_project = pl.pallas_call(
    _proj_body,
    grid=(TC_GRID,),
    in_specs=[
        pl.BlockSpec((DIM, TC_BN), lambda i: (0, i)),
        pl.BlockSpec((DIM, PDIM), lambda i: (0, 0)),
    ],
    out_specs=pl.BlockSpec((PDIM, TC_BN), lambda i: (0, i)),
    out_shape=jax.ShapeDtypeStruct((PDIM, VOCAB), jnp.float32),
)


@functools.partial(
    pl.kernel,
    mesh=_mesh,
    compiler_params=pltpu.CompilerParams(
        use_tc_tiling_on_sc=False, needs_layout_passes=False),
    out_type=jax.ShapeDtypeStruct((VOCAB * PT,), jnp.float32),
    scratch_types=[
        pltpu.VMEM((NCLS * TCH,), jnp.float32),
        pltpu.VMEM((PT * TCH,), jnp.float32),
    ],
)
def _transpose_kernel(ptab_hbm, out_hbm, strip_v, rows_v):
    wid = lax.axis_index("s") * NC + lax.axis_index("c")
    base = lax.min(wid * TSEG, VOCAB - TSEG)
    lane = lax.iota(jnp.int32, 16)
    ibase = (lane % 4) * TCH + lane // 4
    sbase = (lane // 4) * PT + lane % 4

    def chunk_body(ci, carry):
        tok0 = base + lax.min(ci * TCH, TSEG - TCH)
        for c in range(NCLS):
            pltpu.sync_copy(ptab_hbm.at[c, pl.ds(tok0, TCH)],
                            strip_v.at[pl.ds(c * TCH, TCH)])

        def step_body(i, carry2):
            for u in range(4):
                val = plsc.load_gather(strip_v, [ibase + (16 * i + 4 * u)])
                plsc.store_scatter(rows_v, [sbase + (256 * i + 64 * u)], val)
            return carry2

        lax.fori_loop(0, TSTEP, step_body, 0)
        pltpu.sync_copy(rows_v, out_hbm.at[pl.ds(tok0 * PT, PT * TCH)])
        return carry

    lax.fori_loop(0, TNCH, chunk_body, 0)


@functools.partial(
    pl.kernel,
    mesh=_mesh,
    compiler_params=pltpu.CompilerParams(
        use_tc_tiling_on_sc=False, needs_layout_passes=False),
    out_type=jax.ShapeDtypeStruct((BATCH * NCLS,), jnp.float32),
    scratch_types=[
        pltpu.VMEM((NBUF, CI), jnp.int32),
        pltpu.VMEM((NBUF, CI, PT), jnp.float32),
        pltpu.VMEM((BPW * NCLS,), jnp.float32),
        pltpu.VMEM((16,), jnp.float32),
        pltpu.VMEM((64,), jnp.float32),
        pltpu.SemaphoreType.DMA,
    ],
)
def _bag_kernel(text_hbm, ptab_hbm, fcb_hbm, out_hbm,
                idx_v, rows_v, out_v, fcb_v, stage_v, sem):
    wid = lax.axis_index("s") * NC + lax.axis_index("c")
    base = wid * BPW

    pltpu.sync_copy(fcb_hbm, fcb_v)
    inv = jnp.float32(1.0 / SEQ)
    bias = fcb_v[...]
    lane = lax.iota(jnp.int32, 16)
    gidx = (lane // 4) * PT + (lane % 4)

    def fetch(g, k):
        row0 = base + g * CL
        pltpu.sync_copy(text_hbm.at[pl.ds(row0 * SEQ, CI)], idx_v.at[k])
        for off, ln in CHUNKS:
            pltpu.async_copy(
                ptab_hbm.at[idx_v.at[k, pl.ds(off, ln)]],
                rows_v.at[k, pl.ds(off, ln)],
                sem,
            )

    def drain(k):
        for off, ln in CHUNKS:
            pltpu.make_async_copy(
                ptab_hbm.at[idx_v.at[k, pl.ds(off, ln)]],
                rows_v.at[k, pl.ds(off, ln)],
                sem,
            ).wait()

    for g in range(NBUF):
        fetch(g, g)

    def cluster_body(g, carry):
        k = lax.rem(g, NBUF)
        drain(k)

        for q in range(CL // 4):
            for b in range(4):
                r0 = (q * 4 + b) * SEQ

                def acc_body(i, accs):
                    a0, a1, a2, a3 = accs
                    r = r0 + i * 8
                    a0 = a0 + rows_v[k, r, pl.ds(0, 16)]
                    a1 = a1 + rows_v[k, r + 1, pl.ds(0, 16)]
                    a2 = a2 + rows_v[k, r + 2, pl.ds(0, 16)]
                    a3 = a3 + rows_v[k, r + 3, pl.ds(0, 16)]
                    a0 = a0 + rows_v[k, r + 4, pl.ds(0, 16)]
                    a1 = a1 + rows_v[k, r + 5, pl.ds(0, 16)]
                    a2 = a2 + rows_v[k, r + 6, pl.ds(0, 16)]
                    a3 = a3 + rows_v[k, r + 7, pl.ds(0, 16)]
                    return (a0, a1, a2, a3)

                z = jnp.zeros((16,), jnp.float32)
                a0, a1, a2, a3 = lax.fori_loop(0, SEQ // 8, acc_body,
                                               (z, z, z, z))
                stage_v[pl.ds(b * 16, 16)] = (a0 + a1) + (a2 + a3)

            res = plsc.load_gather(stage_v, [gidx])
            out_v[pl.ds((g * CL + q * 4) * NCLS, 16)] = res * inv + bias

        gn = lax.min(g + NBUF, NCLUST - 1)
        fetch(gn, k)
        return carry

    lax.fori_loop(0, NCLUST, cluster_body, 0)
    drain(lax.rem(NCLUST, NBUF))
    drain(lax.rem(NCLUST + 1, NBUF))
    pltpu.sync_copy(out_v, out_hbm.at[pl.ds(base * NCLS, BPW * NCLS)])


def kernel(text, emb_table, fc_w, fc_b):
    text_flat = text.reshape(-1).astype(jnp.int32)
    wpad = jnp.zeros((DIM, PDIM), jnp.float32).at[:, :NCLS].set(
        fc_w.astype(jnp.float32).T)
    ptab_cm = _project(emb_table.T, wpad)
    ptab = _transpose_kernel(ptab_cm).reshape(VOCAB, PT)
    fcb_tiled = jnp.tile(fc_b.astype(jnp.float32), 4)
    out = _bag_kernel(text_flat, ptab, fcb_tiled)
    return out.reshape(BATCH, NCLS)


# speedup vs baseline: 3.8154x; 2.5615x over previous
import functools

import jax
import jax.numpy as jnp
from jax import lax
from jax.experimental import pallas as pl
from jax.experimental.pallas import tpu as pltpu
from jax.experimental.pallas import tpu_sc as plsc

BATCH = 16384
SEQ = 200
DIM = 64
NCLS = 4
PDIM = 8
PT = 16

NC = 2
NS = 16
NW = NC * NS
BPW = BATCH // NW
CL = 8
CI = CL * SEQ
NCLUST = BPW // CL
NBUF = 2
CHUNKS = [(o, min(128, CI - o)) for o in range(0, CI, 128)]

VOCAB = 1000000
TC_BN = 16384
VOCABP = 1015808
TC_GRID = VOCABP // TC_BN
TC_BNB = TC_BN // 128

TSEG = VOCABP // 32
TCH = 3200
TNCH = -(-TSEG // TCH)
TSTEP = TCH // 16

_mesh = plsc.VectorSubcoreMesh(core_axis_name="c", subcore_axis_name="s")


def _proj_body(t_ref, w_ref, o_ref):
    p = jax.lax.dot_general(w_ref[...], t_ref[...],
                            (((0,), (0,)), ((), ())),
                            preferred_element_type=jnp.float32)
    o_ref[...] = p.reshape(PDIM, TC_BNB, 128)


_project = pl.pallas_call(
    _proj_body,
    grid=(TC_GRID,),
    in_specs=[
        pl.BlockSpec((DIM, TC_BN), lambda i: (0, i)),
        pl.BlockSpec((DIM, PDIM), lambda i: (0, 0)),
    ],
    out_specs=pl.BlockSpec((PDIM, TC_BNB, 128), lambda i: (0, i, 0)),
    out_shape=jax.ShapeDtypeStruct((PDIM, VOCABP // 128, 128), jnp.float32),
)


@functools.partial(
    pl.kernel,
    mesh=_mesh,
    compiler_params=pltpu.CompilerParams(
        use_tc_tiling_on_sc=False, needs_layout_passes=False),
    out_type=jax.ShapeDtypeStruct((VOCABP * PT,), jnp.float32),
    scratch_types=[
        pltpu.VMEM((NCLS * TCH,), jnp.float32),
        pltpu.VMEM((PT * TCH,), jnp.float32),
    ],
)
def _transpose_kernel(ptab_hbm, out_hbm, strip_v, rows_v):
    wid = lax.axis_index("s") * NC + lax.axis_index("c")
    base = wid * TSEG
    lane = lax.iota(jnp.int32, 16)
    ibase = (lane % 4) * TCH + lane // 4
    sbase = (lane // 4) * PT + lane % 4

    def chunk_body(ci, carry):
        tok0 = base + lax.min(ci * TCH, TSEG - TCH)
        for c in range(NCLS):
            pltpu.sync_copy(ptab_hbm.at[c, pl.ds(tok0, TCH)],
                            strip_v.at[pl.ds(c * TCH, TCH)])

        def step_body(i, carry2):
            for u in range(4):
                val = plsc.load_gather(strip_v, [ibase + (16 * i + 4 * u)])
                plsc.store_scatter(rows_v, [sbase + (256 * i + 64 * u)], val)
            return carry2

        lax.fori_loop(0, TSTEP, step_body, 0)
        pltpu.sync_copy(rows_v, out_hbm.at[pl.ds(tok0 * PT, PT * TCH)])
        return carry

    lax.fori_loop(0, TNCH, chunk_body, 0)


@functools.partial(
    pl.kernel,
    mesh=_mesh,
    compiler_params=pltpu.CompilerParams(
        use_tc_tiling_on_sc=False, needs_layout_passes=False),
    out_type=jax.ShapeDtypeStruct((BATCH * NCLS,), jnp.float32),
    scratch_types=[
        pltpu.VMEM((NBUF, CI), jnp.int32),
        pltpu.VMEM((NBUF, CI, PT), jnp.float32),
        pltpu.VMEM((BPW * NCLS,), jnp.float32),
        pltpu.VMEM((16,), jnp.float32),
        pltpu.VMEM((64,), jnp.float32),
        pltpu.SemaphoreType.DMA,
    ],
)
def _bag_kernel(text_hbm, ptab_hbm, fcb_hbm, out_hbm,
                idx_v, rows_v, out_v, fcb_v, stage_v, sem):
    wid = lax.axis_index("s") * NC + lax.axis_index("c")
    base = wid * BPW

    pltpu.sync_copy(fcb_hbm, fcb_v)
    inv = jnp.float32(1.0 / SEQ)
    bias = fcb_v[...]
    lane = lax.iota(jnp.int32, 16)
    gidx = (lane // 4) * PT + (lane % 4)

    def fetch(g, k):
        row0 = base + g * CL
        pltpu.sync_copy(text_hbm.at[pl.ds(row0 * SEQ, CI)], idx_v.at[k])
        for off, ln in CHUNKS:
            pltpu.async_copy(
                ptab_hbm.at[idx_v.at[k, pl.ds(off, ln)]],
                rows_v.at[k, pl.ds(off, ln)],
                sem,
            )

    def drain(k):
        for off, ln in CHUNKS:
            pltpu.make_async_copy(
                ptab_hbm.at[idx_v.at[k, pl.ds(off, ln)]],
                rows_v.at[k, pl.ds(off, ln)],
                sem,
            ).wait()

    for g in range(NBUF):
        fetch(g, g)

    def cluster_body(g, carry):
        k = lax.rem(g, NBUF)
        drain(k)

        for q in range(CL // 4):
            for b in range(4):
                r0 = (q * 4 + b) * SEQ

                def acc_body(i, accs):
                    a0, a1, a2, a3 = accs
                    r = r0 + i * 8
                    a0 = a0 + rows_v[k, r, pl.ds(0, 16)]
                    a1 = a1 + rows_v[k, r + 1, pl.ds(0, 16)]
                    a2 = a2 + rows_v[k, r + 2, pl.ds(0, 16)]
                    a3 = a3 + rows_v[k, r + 3, pl.ds(0, 16)]
                    a0 = a0 + rows_v[k, r + 4, pl.ds(0, 16)]
                    a1 = a1 + rows_v[k, r + 5, pl.ds(0, 16)]
                    a2 = a2 + rows_v[k, r + 6, pl.ds(0, 16)]
                    a3 = a3 + rows_v[k, r + 7, pl.ds(0, 16)]
                    return (a0, a1, a2, a3)

                z = jnp.zeros((16,), jnp.float32)
                a0, a1, a2, a3 = lax.fori_loop(0, SEQ // 8, acc_body,
                                               (z, z, z, z))
                stage_v[pl.ds(b * 16, 16)] = (a0 + a1) + (a2 + a3)

            res = plsc.load_gather(stage_v, [gidx])
            out_v[pl.ds((g * CL + q * 4) * NCLS, 16)] = res * inv + bias

        gn = lax.min(g + NBUF, NCLUST - 1)
        fetch(gn, k)
        return carry

    lax.fori_loop(0, NCLUST, cluster_body, 0)
    drain(lax.rem(NCLUST, NBUF))
    drain(lax.rem(NCLUST + 1, NBUF))
    pltpu.sync_copy(out_v, out_hbm.at[pl.ds(base * NCLS, BPW * NCLS)])


def kernel(text, emb_table, fc_w, fc_b):
    text_flat = text.reshape(-1).astype(jnp.int32)
    wpad = jnp.zeros((DIM, PDIM), jnp.float32).at[:, :NCLS].set(
        fc_w.astype(jnp.float32).T)
    ptab_cm = _project(emb_table.T, wpad).reshape(PDIM, VOCABP)
    ptab = _transpose_kernel(ptab_cm).reshape(VOCABP, PT)
    fcb_tiled = jnp.tile(fc_b.astype(jnp.float32), 4)
    out = _bag_kernel(text_flat, ptab, fcb_tiled)
    return out.reshape(BATCH, NCLS)


# speedup vs baseline: 4.3451x; 1.1388x over previous
import functools

import jax
import jax.numpy as jnp
from jax import lax
from jax.experimental import pallas as pl
from jax.experimental.pallas import tpu as pltpu
from jax.experimental.pallas import tpu_sc as plsc

BATCH = 16384
SEQ = 200
DIM = 64
NCLS = 4
PDIM = 8
PT = 16

NC = 2
NS = 16
NW = NC * NS
BPW = BATCH // NW
CL = 8
CI = CL * SEQ
NCLUST = BPW // CL
NBUF = 2
CHUNKS = [(o, min(128, CI - o)) for o in range(0, CI, 128)]

VOCAB = 1000000
TC_BN = 16384
VOCABP = 1015808
TC_GRID = VOCABP // TC_BN
TC_BNB = TC_BN // 128

TSEG = VOCABP // 32
TCH = 1600
TNCH = -(-TSEG // TCH)
TSTEP = TCH // 16

_mesh = plsc.VectorSubcoreMesh(core_axis_name="c", subcore_axis_name="s")


def _proj_body(t_ref, w_ref, o_ref):
    p = jax.lax.dot_general(w_ref[...], t_ref[...],
                            (((0,), (0,)), ((), ())),
                            preferred_element_type=jnp.float32)
    o_ref[...] = p.reshape(PDIM, TC_BNB, 128)


_project = pl.pallas_call(
    _proj_body,
    grid=(TC_GRID,),
    in_specs=[
        pl.BlockSpec((DIM, TC_BN), lambda i: (0, i)),
        pl.BlockSpec((DIM, PDIM), lambda i: (0, 0)),
    ],
    out_specs=pl.BlockSpec((PDIM, TC_BNB, 128), lambda i: (0, i, 0)),
    out_shape=jax.ShapeDtypeStruct((PDIM, VOCABP // 128, 128), jnp.float32),
)


@functools.partial(
    pl.kernel,
    mesh=_mesh,
    compiler_params=pltpu.CompilerParams(
        use_tc_tiling_on_sc=False, needs_layout_passes=False),
    out_type=jax.ShapeDtypeStruct((VOCABP * PT,), jnp.float32),
    scratch_types=[
        pltpu.VMEM((NBUF, NCLS * TCH), jnp.float32),
        pltpu.VMEM((NBUF, PT * TCH), jnp.float32),
        pltpu.SemaphoreType.DMA,
        pltpu.SemaphoreType.DMA,
    ],
)
def _transpose_kernel(ptab_hbm, out_hbm, strip_v, rows_v, sem_i, sem_o):
    wid = lax.axis_index("s") * NC + lax.axis_index("c")
    base = wid * TSEG
    lane = lax.iota(jnp.int32, 16)
    ibase = (lane % 4) * TCH + lane // 4
    sbase = (lane // 4) * PT + lane % 4

    def tok_of(ci):
        return base + lax.min(ci * TCH, TSEG - TCH)

    def fetch(ci, k):
        tok0 = tok_of(ci)
        for c in range(NCLS):
            pltpu.async_copy(ptab_hbm.at[c, pl.ds(tok0, TCH)],
                             strip_v.at[k, pl.ds(c * TCH, TCH)], sem_i)

    def drain(ci, k):
        tok0 = tok_of(ci)
        for c in range(NCLS):
            pltpu.make_async_copy(ptab_hbm.at[c, pl.ds(tok0, TCH)],
                                  strip_v.at[k, pl.ds(c * TCH, TCH)],
                                  sem_i).wait()

    for ci in range(NBUF):
        fetch(ci, ci)

    def chunk_body(ci, carry):
        k = lax.rem(ci, NBUF)
        tok0 = tok_of(ci)
        drain(ci, k)

        @pl.when(ci >= NBUF)
        def _():
            pltpu.make_async_copy(
                rows_v.at[k], out_hbm.at[pl.ds(tok0 * PT, PT * TCH)],
                sem_o).wait()

        def step_body(i, carry2):
            for u in range(4):
                val = plsc.load_gather(
                    strip_v.at[k], [ibase + (16 * i + 4 * u)])
                plsc.store_scatter(
                    rows_v.at[k], [sbase + (256 * i + 64 * u)], val)
            return carry2

        lax.fori_loop(0, TSTEP, step_body, 0)
        pltpu.async_copy(rows_v.at[k],
                         out_hbm.at[pl.ds(tok0 * PT, PT * TCH)], sem_o)
        fetch(lax.min(ci + NBUF, TNCH - 1), k)
        return carry

    lax.fori_loop(0, TNCH, chunk_body, 0)
    for ci in range(NBUF):
        drain(TNCH - 1, lax.rem(TNCH + ci, NBUF))
        pltpu.make_async_copy(
            rows_v.at[lax.rem(TNCH + ci, NBUF)],
            out_hbm.at[pl.ds(tok_of(TNCH - 2 + ci) * PT, PT * TCH)],
            sem_o).wait()


@functools.partial(
    pl.kernel,
    mesh=_mesh,
    compiler_params=pltpu.CompilerParams(
        use_tc_tiling_on_sc=False, needs_layout_passes=False),
    out_type=jax.ShapeDtypeStruct((BATCH * NCLS,), jnp.float32),
    scratch_types=[
        pltpu.VMEM((NBUF, CI), jnp.int32),
        pltpu.VMEM((NBUF, CI, PT), jnp.float32),
        pltpu.VMEM((BPW * NCLS,), jnp.float32),
        pltpu.VMEM((16,), jnp.float32),
        pltpu.VMEM((64,), jnp.float32),
        pltpu.SemaphoreType.DMA,
    ],
)
def _bag_kernel(text_hbm, ptab_hbm, fcb_hbm, out_hbm,
                idx_v, rows_v, out_v, fcb_v, stage_v, sem):
    wid = lax.axis_index("s") * NC + lax.axis_index("c")
    base = wid * BPW

    pltpu.sync_copy(fcb_hbm, fcb_v)
    inv = jnp.float32(1.0 / SEQ)
    bias = fcb_v[...]
    lane = lax.iota(jnp.int32, 16)
    gidx = (lane // 4) * PT + (lane % 4)

    def fetch(g, k):
        row0 = base + g * CL
        pltpu.sync_copy(text_hbm.at[pl.ds(row0 * SEQ, CI)], idx_v.at[k])
        for off, ln in CHUNKS:
            pltpu.async_copy(
                ptab_hbm.at[idx_v.at[k, pl.ds(off, ln)]],
                rows_v.at[k, pl.ds(off, ln)],
                sem,
            )

    def drain(k):
        for off, ln in CHUNKS:
            pltpu.make_async_copy(
                ptab_hbm.at[idx_v.at[k, pl.ds(off, ln)]],
                rows_v.at[k, pl.ds(off, ln)],
                sem,
            ).wait()

    for g in range(NBUF):
        fetch(g, g)

    def cluster_body(g, carry):
        k = lax.rem(g, NBUF)
        drain(k)

        for q in range(CL // 4):
            for b in range(4):
                r0 = (q * 4 + b) * SEQ

                def acc_body(i, accs):
                    a0, a1, a2, a3 = accs
                    r = r0 + i * 8
                    a0 = a0 + rows_v[k, r, pl.ds(0, 16)]
                    a1 = a1 + rows_v[k, r + 1, pl.ds(0, 16)]
                    a2 = a2 + rows_v[k, r + 2, pl.ds(0, 16)]
                    a3 = a3 + rows_v[k, r + 3, pl.ds(0, 16)]
                    a0 = a0 + rows_v[k, r + 4, pl.ds(0, 16)]
                    a1 = a1 + rows_v[k, r + 5, pl.ds(0, 16)]
                    a2 = a2 + rows_v[k, r + 6, pl.ds(0, 16)]
                    a3 = a3 + rows_v[k, r + 7, pl.ds(0, 16)]
                    return (a0, a1, a2, a3)

                z = jnp.zeros((16,), jnp.float32)
                a0, a1, a2, a3 = lax.fori_loop(0, SEQ // 8, acc_body,
                                               (z, z, z, z))
                stage_v[pl.ds(b * 16, 16)] = (a0 + a1) + (a2 + a3)

            res = plsc.load_gather(stage_v, [gidx])
            out_v[pl.ds((g * CL + q * 4) * NCLS, 16)] = res * inv + bias

        gn = lax.min(g + NBUF, NCLUST - 1)
        fetch(gn, k)
        return carry

    lax.fori_loop(0, NCLUST, cluster_body, 0)
    drain(lax.rem(NCLUST, NBUF))
    drain(lax.rem(NCLUST + 1, NBUF))
    pltpu.sync_copy(out_v, out_hbm.at[pl.ds(base * NCLS, BPW * NCLS)])


def kernel(text, emb_table, fc_w, fc_b):
    text_flat = text.reshape(-1).astype(jnp.int32)
    wpad = jnp.zeros((DIM, PDIM), jnp.float32).at[:, :NCLS].set(
        fc_w.astype(jnp.float32).T)
    ptab_cm = _project(emb_table.T, wpad).reshape(PDIM, VOCABP)
    ptab = _transpose_kernel(ptab_cm).reshape(VOCABP, PT)
    fcb_tiled = jnp.tile(fc_b.astype(jnp.float32), 4)
    out = _bag_kernel(text_flat, ptab, fcb_tiled)
    return out.reshape(BATCH, NCLS)


# speedup vs baseline: 4.4626x; 1.0271x over previous
import functools

import jax
import jax.numpy as jnp
from jax import lax
from jax.experimental import pallas as pl
from jax.experimental.pallas import tpu as pltpu
from jax.experimental.pallas import tpu_sc as plsc

BATCH = 16384
SEQ = 200
DIM = 64
NCLS = 4
PDIM = 8
PT = 16

NC = 2
NS = 16
NW = NC * NS
BPW = BATCH // NW
CL = 8
CI = CL * SEQ
NCLUST = BPW // CL
NBUF = 2
CHUNKS = [(0, CI)]

VOCAB = 1000000
TC_BN = 32768
VOCABP = 1015808
TC_GRID = VOCABP // TC_BN
TC_BNB = TC_BN // 128

TSEG = VOCABP // 32
TCH = 1600
TNCH = -(-TSEG // TCH)
TSTEP = TCH // 16

_mesh = plsc.VectorSubcoreMesh(core_axis_name="c", subcore_axis_name="s")


def _proj_body(t_ref, w_ref, o_ref):
    p = jax.lax.dot_general(w_ref[...], t_ref[...],
                            (((0,), (0,)), ((), ())),
                            preferred_element_type=jnp.float32)
    o_ref[...] = p.reshape(PDIM, TC_BNB, 128)


_project = pl.pallas_call(
    _proj_body,
    grid=(TC_GRID,),
    in_specs=[
        pl.BlockSpec((DIM, TC_BN), lambda i: (0, i)),
        pl.BlockSpec((DIM, PDIM), lambda i: (0, 0)),
    ],
    out_specs=pl.BlockSpec((PDIM, TC_BNB, 128), lambda i: (0, i, 0)),
    out_shape=jax.ShapeDtypeStruct((PDIM, VOCABP // 128, 128), jnp.float32),
)


@functools.partial(
    pl.kernel,
    mesh=_mesh,
    compiler_params=pltpu.CompilerParams(
        use_tc_tiling_on_sc=False, needs_layout_passes=False),
    out_type=jax.ShapeDtypeStruct((VOCABP * PT,), jnp.float32),
    scratch_types=[
        pltpu.VMEM((NBUF, NCLS * TCH), jnp.float32),
        pltpu.VMEM((NBUF, PT * TCH), jnp.float32),
        pltpu.SemaphoreType.DMA,
        pltpu.SemaphoreType.DMA,
    ],
)
def _transpose_kernel(ptab_hbm, out_hbm, strip_v, rows_v, sem_i, sem_o):
    wid = lax.axis_index("s") * NC + lax.axis_index("c")
    base = wid * TSEG
    lane = lax.iota(jnp.int32, 16)
    ibase = (lane % 4) * TCH + lane // 4
    sbase = (lane // 4) * PT + lane % 4

    def tok_of(ci):
        return base + lax.min(ci * TCH, TSEG - TCH)

    def fetch(ci, k):
        tok0 = tok_of(ci)
        for c in range(NCLS):
            pltpu.async_copy(ptab_hbm.at[c, pl.ds(tok0, TCH)],
                             strip_v.at[k, pl.ds(c * TCH, TCH)], sem_i)

    def drain(ci, k):
        tok0 = tok_of(ci)
        for c in range(NCLS):
            pltpu.make_async_copy(ptab_hbm.at[c, pl.ds(tok0, TCH)],
                                  strip_v.at[k, pl.ds(c * TCH, TCH)],
                                  sem_i).wait()

    for ci in range(NBUF):
        fetch(ci, ci)

    def chunk_body(ci, carry):
        k = lax.rem(ci, NBUF)
        tok0 = tok_of(ci)
        drain(ci, k)

        @pl.when(ci >= NBUF)
        def _():
            pltpu.make_async_copy(
                rows_v.at[k], out_hbm.at[pl.ds(tok0 * PT, PT * TCH)],
                sem_o).wait()

        def step_body(i, carry2):
            for u in range(4):
                val = plsc.load_gather(
                    strip_v.at[k], [ibase + (16 * i + 4 * u)])
                plsc.store_scatter(
                    rows_v.at[k], [sbase + (256 * i + 64 * u)], val)
            return carry2

        lax.fori_loop(0, TSTEP, step_body, 0)
        pltpu.async_copy(rows_v.at[k],
                         out_hbm.at[pl.ds(tok0 * PT, PT * TCH)], sem_o)
        fetch(lax.min(ci + NBUF, TNCH - 1), k)
        return carry

    lax.fori_loop(0, TNCH, chunk_body, 0)
    for ci in range(NBUF):
        drain(TNCH - 1, lax.rem(TNCH + ci, NBUF))
        pltpu.make_async_copy(
            rows_v.at[lax.rem(TNCH + ci, NBUF)],
            out_hbm.at[pl.ds(tok_of(TNCH - 2 + ci) * PT, PT * TCH)],
            sem_o).wait()


@functools.partial(
    pl.kernel,
    mesh=_mesh,
    compiler_params=pltpu.CompilerParams(
        use_tc_tiling_on_sc=False, needs_layout_passes=False),
    out_type=jax.ShapeDtypeStruct((BATCH * NCLS,), jnp.float32),
    scratch_types=[
        pltpu.VMEM((NBUF, CI), jnp.int32),
        pltpu.VMEM((NBUF, CI, PT), jnp.float32),
        pltpu.VMEM((BPW * NCLS,), jnp.float32),
        pltpu.VMEM((16,), jnp.float32),
        pltpu.VMEM((64,), jnp.float32),
        pltpu.SemaphoreType.DMA,
    ],
)
def _bag_kernel(text_hbm, ptab_hbm, fcb_hbm, out_hbm,
                idx_v, rows_v, out_v, fcb_v, stage_v, sem):
    wid = lax.axis_index("s") * NC + lax.axis_index("c")
    base = wid * BPW

    pltpu.sync_copy(fcb_hbm, fcb_v)
    inv = jnp.float32(1.0 / SEQ)
    bias = fcb_v[...]
    lane = lax.iota(jnp.int32, 16)
    gidx = (lane // 4) * PT + (lane % 4)

    def fetch(g, k):
        row0 = base + g * CL
        pltpu.sync_copy(text_hbm.at[pl.ds(row0 * SEQ, CI)], idx_v.at[k])
        for off, ln in CHUNKS:
            pltpu.async_copy(
                ptab_hbm.at[idx_v.at[k, pl.ds(off, ln)]],
                rows_v.at[k, pl.ds(off, ln)],
                sem,
            )

    def drain(k):
        for off, ln in CHUNKS:
            pltpu.make_async_copy(
                ptab_hbm.at[idx_v.at[k, pl.ds(off, ln)]],
                rows_v.at[k, pl.ds(off, ln)],
                sem,
            ).wait()

    for g in range(NBUF):
        fetch(g, g)

    def cluster_body(g, carry):
        k = lax.rem(g, NBUF)
        drain(k)

        for q in range(CL // 4):
            for b in range(4):
                r0 = (q * 4 + b) * SEQ

                def acc_body(i, accs):
                    a0, a1, a2, a3 = accs
                    r = r0 + i * 8
                    a0 = a0 + rows_v[k, r, pl.ds(0, 16)]
                    a1 = a1 + rows_v[k, r + 1, pl.ds(0, 16)]
                    a2 = a2 + rows_v[k, r + 2, pl.ds(0, 16)]
                    a3 = a3 + rows_v[k, r + 3, pl.ds(0, 16)]
                    a0 = a0 + rows_v[k, r + 4, pl.ds(0, 16)]
                    a1 = a1 + rows_v[k, r + 5, pl.ds(0, 16)]
                    a2 = a2 + rows_v[k, r + 6, pl.ds(0, 16)]
                    a3 = a3 + rows_v[k, r + 7, pl.ds(0, 16)]
                    return (a0, a1, a2, a3)

                z = jnp.zeros((16,), jnp.float32)
                a0, a1, a2, a3 = lax.fori_loop(0, SEQ // 8, acc_body,
                                               (z, z, z, z))
                stage_v[pl.ds(b * 16, 16)] = (a0 + a1) + (a2 + a3)

            res = plsc.load_gather(stage_v, [gidx])
            out_v[pl.ds((g * CL + q * 4) * NCLS, 16)] = res * inv + bias

        gn = lax.min(g + NBUF, NCLUST - 1)
        fetch(gn, k)
        return carry

    lax.fori_loop(0, NCLUST, cluster_body, 0)
    drain(lax.rem(NCLUST, NBUF))
    drain(lax.rem(NCLUST + 1, NBUF))
    pltpu.sync_copy(out_v, out_hbm.at[pl.ds(base * NCLS, BPW * NCLS)])


def kernel(text, emb_table, fc_w, fc_b):
    text_flat = text.reshape(-1).astype(jnp.int32)
    wpad = jnp.zeros((DIM, PDIM), jnp.float32).at[:, :NCLS].set(
        fc_w.astype(jnp.float32).T)
    ptab_cm = _project(emb_table.T, wpad).reshape(PDIM, VOCABP)
    ptab = _transpose_kernel(ptab_cm).reshape(VOCABP, PT)
    fcb_tiled = jnp.tile(fc_b.astype(jnp.float32), 4)
    out = _bag_kernel(text_flat, ptab, fcb_tiled)
    return out.reshape(BATCH, NCLS)


# speedup vs baseline: 4.9576x; 1.1109x over previous
import functools

import jax
import jax.numpy as jnp
from jax import lax
from jax.experimental import pallas as pl
from jax.experimental.pallas import tpu as pltpu
from jax.experimental.pallas import tpu_sc as plsc

BATCH = 16384
SEQ = 200
DIM = 64
NCLS = 4
PDIM = 8
PT = 16

NC = 2
NS = 16
NW = NC * NS
BPW = BATCH // NW
CL = 8
CI = CL * SEQ
NCLUST = BPW // CL
NBUF = 2
GBUF = 3
CHUNKS = [(0, CI)]

VOCAB = 1000000
TC_BN = 32768
VOCABP = 1015808
TC_GRID = VOCABP // TC_BN
TC_BNB = TC_BN // 128

TSEG = VOCABP // 32
TCH = 1600
TNCH = -(-TSEG // TCH)
TSTEP = TCH // 16

_mesh = plsc.VectorSubcoreMesh(core_axis_name="c", subcore_axis_name="s")


def _proj_body(t_ref, w_ref, o_ref):
    p = jax.lax.dot_general(w_ref[...], t_ref[...],
                            (((0,), (0,)), ((), ())),
                            preferred_element_type=jnp.float32)
    o_ref[...] = p.reshape(PDIM, TC_BNB, 128)


_project = pl.pallas_call(
    _proj_body,
    grid=(TC_GRID,),
    in_specs=[
        pl.BlockSpec((DIM, TC_BN), lambda i: (0, i)),
        pl.BlockSpec((DIM, PDIM), lambda i: (0, 0)),
    ],
    out_specs=pl.BlockSpec((PDIM, TC_BNB, 128), lambda i: (0, i, 0)),
    out_shape=jax.ShapeDtypeStruct((PDIM, VOCABP // 128, 128), jnp.float32),
)


@functools.partial(
    pl.kernel,
    mesh=_mesh,
    compiler_params=pltpu.CompilerParams(
        use_tc_tiling_on_sc=False, needs_layout_passes=False),
    out_type=jax.ShapeDtypeStruct((VOCABP * PT,), jnp.float32),
    scratch_types=[
        pltpu.VMEM((NBUF, NCLS * TCH), jnp.float32),
        pltpu.VMEM((NBUF, PT * TCH), jnp.float32),
        pltpu.SemaphoreType.DMA,
        pltpu.SemaphoreType.DMA,
    ],
)
def _transpose_kernel(ptab_hbm, out_hbm, strip_v, rows_v, sem_i, sem_o):
    wid = lax.axis_index("s") * NC + lax.axis_index("c")
    base = wid * TSEG
    lane = lax.iota(jnp.int32, 16)
    ibase = (lane % 4) * TCH + lane // 4
    sbase = (lane // 4) * PT + lane % 4

    def tok_of(ci):
        return base + lax.min(ci * TCH, TSEG - TCH)

    def fetch(ci, k):
        tok0 = tok_of(ci)
        for c in range(NCLS):
            pltpu.async_copy(ptab_hbm.at[c, pl.ds(tok0, TCH)],
                             strip_v.at[k, pl.ds(c * TCH, TCH)], sem_i)

    def drain(ci, k):
        tok0 = tok_of(ci)
        for c in range(NCLS):
            pltpu.make_async_copy(ptab_hbm.at[c, pl.ds(tok0, TCH)],
                                  strip_v.at[k, pl.ds(c * TCH, TCH)],
                                  sem_i).wait()

    for ci in range(NBUF):
        fetch(ci, ci)

    def chunk_body(ci, carry):
        k = lax.rem(ci, NBUF)
        tok0 = tok_of(ci)
        drain(ci, k)

        @pl.when(ci >= NBUF)
        def _():
            pltpu.make_async_copy(
                rows_v.at[k], out_hbm.at[pl.ds(tok0 * PT, PT * TCH)],
                sem_o).wait()

        def step_body(i, carry2):
            for u in range(4):
                val = plsc.load_gather(
                    strip_v.at[k], [ibase + (16 * i + 4 * u)])
                plsc.store_scatter(
                    rows_v.at[k], [sbase + (256 * i + 64 * u)], val)
            return carry2

        lax.fori_loop(0, TSTEP, step_body, 0)
        pltpu.async_copy(rows_v.at[k],
                         out_hbm.at[pl.ds(tok0 * PT, PT * TCH)], sem_o)
        fetch(lax.min(ci + NBUF, TNCH - 1), k)
        return carry

    lax.fori_loop(0, TNCH, chunk_body, 0)
    for ci in range(NBUF):
        drain(TNCH - 1, lax.rem(TNCH + ci, NBUF))
        pltpu.make_async_copy(
            rows_v.at[lax.rem(TNCH + ci, NBUF)],
            out_hbm.at[pl.ds(tok_of(TNCH - 2 + ci) * PT, PT * TCH)],
            sem_o).wait()


@functools.partial(
    pl.kernel,
    mesh=_mesh,
    compiler_params=pltpu.CompilerParams(
        use_tc_tiling_on_sc=False, needs_layout_passes=False),
    out_type=jax.ShapeDtypeStruct((BATCH * NCLS,), jnp.float32),
    scratch_types=[
        pltpu.VMEM((GBUF, CI), jnp.int32),
        pltpu.VMEM((GBUF, CI, PT), jnp.float32),
        pltpu.VMEM((BPW * NCLS,), jnp.float32),
        pltpu.VMEM((16,), jnp.float32),
        pltpu.VMEM((64,), jnp.float32),
        pltpu.SemaphoreType.DMA,
    ],
)
def _bag_kernel(text_hbm, ptab_hbm, fcb_hbm, out_hbm,
                idx_v, rows_v, out_v, fcb_v, stage_v, sem):
    wid = lax.axis_index("s") * NC + lax.axis_index("c")
    base = wid * BPW

    pltpu.sync_copy(fcb_hbm, fcb_v)
    inv = jnp.float32(1.0 / SEQ)
    bias = fcb_v[...]
    lane = lax.iota(jnp.int32, 16)
    gidx = (lane // 4) * PT + (lane % 4)

    def fetch(g, k):
        row0 = base + g * CL
        pltpu.sync_copy(text_hbm.at[pl.ds(row0 * SEQ, CI)], idx_v.at[k])
        for off, ln in CHUNKS:
            pltpu.async_copy(
                ptab_hbm.at[idx_v.at[k, pl.ds(off, ln)]],
                rows_v.at[k, pl.ds(off, ln)],
                sem,
            )

    def drain(k):
        for off, ln in CHUNKS:
            pltpu.make_async_copy(
                ptab_hbm.at[idx_v.at[k, pl.ds(off, ln)]],
                rows_v.at[k, pl.ds(off, ln)],
                sem,
            ).wait()

    for g in range(GBUF):
        fetch(g, g)

    def cluster_body(g, carry):
        k = lax.rem(g, GBUF)
        drain(k)

        for q in range(CL // 4):
            for b in range(4):
                r0 = (q * 4 + b) * SEQ

                def acc_body(i, accs):
                    a0, a1, a2, a3 = accs
                    r = r0 + i * 8
                    a0 = a0 + rows_v[k, r, pl.ds(0, 16)]
                    a1 = a1 + rows_v[k, r + 1, pl.ds(0, 16)]
                    a2 = a2 + rows_v[k, r + 2, pl.ds(0, 16)]
                    a3 = a3 + rows_v[k, r + 3, pl.ds(0, 16)]
                    a0 = a0 + rows_v[k, r + 4, pl.ds(0, 16)]
                    a1 = a1 + rows_v[k, r + 5, pl.ds(0, 16)]
                    a2 = a2 + rows_v[k, r + 6, pl.ds(0, 16)]
                    a3 = a3 + rows_v[k, r + 7, pl.ds(0, 16)]
                    return (a0, a1, a2, a3)

                z = jnp.zeros((16,), jnp.float32)
                a0, a1, a2, a3 = lax.fori_loop(0, SEQ // 8, acc_body,
                                               (z, z, z, z))
                stage_v[pl.ds(b * 16, 16)] = (a0 + a1) + (a2 + a3)

            res = plsc.load_gather(stage_v, [gidx])
            out_v[pl.ds((g * CL + q * 4) * NCLS, 16)] = res * inv + bias

        gn = lax.min(g + GBUF, NCLUST - 1)
        fetch(gn, k)
        return carry

    lax.fori_loop(0, NCLUST, cluster_body, 0)
    for i in range(GBUF):
        drain(lax.rem(NCLUST + i, GBUF))
    pltpu.sync_copy(out_v, out_hbm.at[pl.ds(base * NCLS, BPW * NCLS)])


def kernel(text, emb_table, fc_w, fc_b):
    text_flat = text.reshape(-1).astype(jnp.int32)
    wpad = jnp.zeros((DIM, PDIM), jnp.float32).at[:, :NCLS].set(
        fc_w.astype(jnp.float32).T)
    ptab_cm = _project(emb_table.T, wpad).reshape(PDIM, VOCABP)
    ptab = _transpose_kernel(ptab_cm).reshape(VOCABP, PT)
    fcb_tiled = jnp.tile(fc_b.astype(jnp.float32), 4)
    out = _bag_kernel(text_flat, ptab, fcb_tiled)
    return out.reshape(BATCH, NCLS)
